# baseline (speedup 1.0000x reference)
import functools

import jax
import jax.numpy as jnp
from jax import lax
from jax.experimental import pallas as pl
from jax.experimental.pallas import tpu as pltpu
from jax.experimental.pallas import tpu_sc as plsc

VOCAB = 100000
HIDDEN = 128
B, S = 1024, 200
EPS = 1e-12

NC, NS = 2, 16
NW = NC * NS
TOK = B * S
PER_W = TOK // NW
CH = 128
ITERS = PER_W // CH
PAIRS = ITERS // 2
HS = HIDDEN // 16
NROW = TOK // CH


def _lane_sum(v, shuf):
    for p in shuf:
        v = v + jnp.take(v, p)
    return v


def _rsqrt_newton(v):
    m = v[0]
    bi = lax.bitcast_convert_type(m, jnp.int32)
    y = lax.bitcast_convert_type(jnp.int32(0x5F3759DF) - (bi >> 1), jnp.float32)
    for _ in range(3):
        y = y * (1.5 - 0.5 * m * y * y)
    return jnp.full((16,), y, jnp.float32)


def _fused_body(ids_hbm, cidx_hbm, word_hbm, ptbl_hbm, gam_hbm, bet_hbm, out_hbm,
                idxall_v, call_v, wrows_a, wrows_b, prows_a, prows_b,
                gam_v, bet_v,
                sem_wa, sem_pa, sem_wb, sem_pb, sem_sa, sem_sb):
    wid = lax.axis_index("s") * NC + lax.axis_index("c")
    base = wid * PER_W
    row0 = wid * ITERS
    lane = jnp.arange(16, dtype=jnp.int32)
    shuf = [lax.rem(lane + k, 16) for k in (1, 2, 4, 8)]

    pltpu.sync_copy(ids_hbm.at[pl.ds(base, PER_W)], idxall_v)
    pltpu.sync_copy(cidx_hbm.at[pl.ds(base, PER_W)], call_v)
    pltpu.sync_copy(gam_hbm, gam_v)
    pltpu.sync_copy(bet_hbm, bet_v)
    gam = [gam_v[pl.ds(s * 16, 16)] for s in range(HS)]
    bet = [bet_v[pl.ds(s * 16, 16)] for s in range(HS)]

    def issue(c, wrows_v, prows_v, sem_w, sem_p, sem_s=None):
        off = base + c * CH
        if sem_s is not None:
            pltpu.make_async_copy(
                wrows_v, out_hbm.at[pl.ds(off, CH)], sem_s).wait()
        loc = pl.multiple_of(c * CH, CH)
        pltpu.async_copy(word_hbm.at[idxall_v.at[pl.ds(loc, CH)]], wrows_v, sem_w)
        pltpu.async_copy(ptbl_hbm.at[call_v.at[pl.ds(loc, CH)]], prows_v, sem_p)

    def compute(c, wrows_v, prows_v, sem_w, sem_p, sem_s):
        off = base + c * CH
        loc = pl.multiple_of(c * CH, CH)
        pltpu.make_async_copy(
            word_hbm.at[idxall_v.at[pl.ds(loc, CH)]], wrows_v, sem_w).wait()
        pltpu.make_async_copy(
            ptbl_hbm.at[call_v.at[pl.ds(loc, CH)]], prows_v, sem_p).wait()

        @plsc.parallel_loop(0, CH, 1, unroll=2)
        def token(t):
            xs = []
            a1 = jnp.zeros((16,), jnp.float32)
            a2 = jnp.zeros((16,), jnp.float32)
            for s in range(HS):
                x = wrows_v[t, pl.ds(s * 16, 16)] + prows_v[t, pl.ds(s * 16, 16)]
                xs.append(x)
                a1 = a1 + x
                a2 = a2 + x * x
            mean = _lane_sum(a1, shuf) * (1.0 / HIDDEN)
            msq = _lane_sum(a2, shuf) * (1.0 / HIDDEN)
            rstd = _rsqrt_newton(msq - mean * mean + EPS)
            for s in range(HS):
                wrows_v[t, pl.ds(s * 16, 16)] = (xs[s] - mean) * rstd * gam[s] + bet[s]

        pltpu.async_copy(wrows_v, out_hbm.at[pl.ds(off, CH)], sem_s)

    issue(0, wrows_a, prows_a, sem_wa, sem_pa)

    def pair(i, carry):
        c0 = 2 * i

        @pl.when(i > 0)
        def _():
            issue(c0 + 1, wrows_b, prows_b, sem_wb, sem_pb, sem_sb)

        @pl.when(i == 0)
        def _():
            issue(c0 + 1, wrows_b, prows_b, sem_wb, sem_pb)

        compute(c0, wrows_a, prows_a, sem_wa, sem_pa, sem_sa)

        @pl.when(i < PAIRS - 1)
        def _():
            issue(c0 + 2, wrows_a, prows_a, sem_wa, sem_pa, sem_sa)

        compute(c0 + 1, wrows_b, prows_b, sem_wb, sem_pb, sem_sb)
        return carry

    lax.fori_loop(0, PAIRS, pair, 0)
    last = base + (ITERS - 2) * CH
    pltpu.make_async_copy(wrows_a, out_hbm.at[pl.ds(last, CH)], sem_sa).wait()
    pltpu.make_async_copy(wrows_b, out_hbm.at[pl.ds(last + CH, CH)], sem_sb).wait()


_fused_call = functools.partial(
    pl.kernel,
    mesh=plsc.VectorSubcoreMesh(core_axis_name="c", subcore_axis_name="s"),
    out_type=jax.ShapeDtypeStruct((TOK, HIDDEN), jnp.float32),
    scratch_types=[
        pltpu.VMEM((PER_W,), jnp.int32),
        pltpu.VMEM((PER_W,), jnp.int32),
        pltpu.VMEM((CH, HIDDEN), jnp.float32),
        pltpu.VMEM((CH, HIDDEN), jnp.float32),
        pltpu.VMEM((CH, HIDDEN), jnp.float32),
        pltpu.VMEM((CH, HIDDEN), jnp.float32),
        pltpu.VMEM((HIDDEN,), jnp.float32),
        pltpu.VMEM((HIDDEN,), jnp.float32),
        pltpu.SemaphoreType.DMA,
        pltpu.SemaphoreType.DMA,
        pltpu.SemaphoreType.DMA,
        pltpu.SemaphoreType.DMA,
        pltpu.SemaphoreType.DMA,
        pltpu.SemaphoreType.DMA,
    ],
)(_fused_body)


def _prep_body(pos_ref, type_ref, tt_ref, ptbl_ref, cidx_ref):
    ptbl_ref[...] = type_ref[...][:, None, :] + pos_ref[...][None, :, :]
    r = lax.broadcasted_iota(jnp.int32, (NROW, CH), 0)
    col = lax.broadcasted_iota(jnp.int32, (NROW, CH), 1)
    pos = lax.rem(r * CH + col, S)
    cidx_ref[...] = tt_ref[...] * S + pos


def kernel(input_ids, token_type_ids, word_emb, pos_emb, type_emb, ln_gamma, ln_beta):
    ids_flat = input_ids.reshape(TOK).astype(jnp.int32)
    tt2d = token_type_ids.reshape(NROW, CH).astype(jnp.int32)

    ptbl, cidx = pl.pallas_call(
        _prep_body,
        out_shape=[
            jax.ShapeDtypeStruct((2, S, HIDDEN), jnp.float32),
            jax.ShapeDtypeStruct((NROW, CH), jnp.int32),
        ],
    )(pos_emb[:S], type_emb, tt2d)

    out = _fused_call(ids_flat, cidx.reshape(TOK), word_emb,
                      ptbl.reshape(2 * S, HIDDEN), ln_gamma, ln_beta)
    return out.reshape(B, S, HIDDEN)


# speedup vs baseline: 1.1089x; 1.1089x over previous
import functools

import jax
import jax.numpy as jnp
from jax import lax
from jax.experimental import pallas as pl
from jax.experimental.pallas import tpu as pltpu
from jax.experimental.pallas import tpu_sc as plsc

VOCAB = 100000
HIDDEN = 128
B, S = 1024, 200
EPS = 1e-12

NC, NS = 2, 16
NW = NC * NS
TOK = B * S
PER_W = TOK // NW
CH = 128
ITERS = PER_W // CH
PAIRS = ITERS // 2
HS = HIDDEN // 16
NROW = TOK // CH


def _lane_sum(v, shuf):
    for p in shuf:
        v = v + jnp.take(v, p)
    return v


def _rsqrt_newton(v):
    m = v[0]
    bi = lax.bitcast_convert_type(m, jnp.int32)
    y = lax.bitcast_convert_type(jnp.int32(0x5F3759DF) - (bi >> 1), jnp.float32)
    for _ in range(3):
        y = y * (1.5 - 0.5 * m * y * y)
    return jnp.full((16,), y, jnp.float32)


def _fused_body(ids_hbm, cidx_hbm, word_hbm, ptbl_hbm, gam_hbm, bet_hbm, out_hbm,
                idxall_v, call_v, wrows_a, wrows_b, prows_a, prows_b,
                gam_v, bet_v, ptbl_sh,
                sem_wa, sem_pa, sem_wb, sem_pb, sem_sa, sem_sb):
    wid = lax.axis_index("s") * NC + lax.axis_index("c")
    base = wid * PER_W
    lane = jnp.arange(16, dtype=jnp.int32)
    shuf = [lax.rem(lane + k, 16) for k in (1, 2, 4, 8)]

    @pl.when(lax.axis_index("s") == 0)
    def _():
        pltpu.sync_copy(ptbl_hbm, ptbl_sh)

    plsc.subcore_barrier()

    pltpu.sync_copy(ids_hbm.at[pl.ds(base, PER_W)], idxall_v)
    pltpu.sync_copy(cidx_hbm.at[pl.ds(base, PER_W)], call_v)
    pltpu.sync_copy(gam_hbm, gam_v)
    pltpu.sync_copy(bet_hbm, bet_v)
    gam = [gam_v[pl.ds(s * 16, 16)] for s in range(HS)]
    bet = [bet_v[pl.ds(s * 16, 16)] for s in range(HS)]

    def issue(c, wrows_v, prows_v, sem_w, sem_p, sem_s=None):
        off = base + c * CH
        if sem_s is not None:
            pltpu.make_async_copy(
                wrows_v, out_hbm.at[pl.ds(off, CH)], sem_s).wait()
        loc = pl.multiple_of(c * CH, CH)
        pltpu.async_copy(word_hbm.at[idxall_v.at[pl.ds(loc, CH)]], wrows_v, sem_w)
        pltpu.async_copy(ptbl_sh.at[call_v.at[pl.ds(loc, CH)]], prows_v, sem_p)

    def compute(c, wrows_v, prows_v, sem_w, sem_p, sem_s):
        off = base + c * CH
        loc = pl.multiple_of(c * CH, CH)
        pltpu.make_async_copy(
            word_hbm.at[idxall_v.at[pl.ds(loc, CH)]], wrows_v, sem_w).wait()
        pltpu.make_async_copy(
            ptbl_sh.at[call_v.at[pl.ds(loc, CH)]], prows_v, sem_p).wait()

        @plsc.parallel_loop(0, CH, 1, unroll=2)
        def token(t):
            xs = []
            a1 = jnp.zeros((16,), jnp.float32)
            a2 = jnp.zeros((16,), jnp.float32)
            for s in range(HS):
                x = wrows_v[t, pl.ds(s * 16, 16)] + prows_v[t, pl.ds(s * 16, 16)]
                xs.append(x)
                a1 = a1 + x
                a2 = a2 + x * x
            mean = _lane_sum(a1, shuf) * (1.0 / HIDDEN)
            msq = _lane_sum(a2, shuf) * (1.0 / HIDDEN)
            rstd = _rsqrt_newton(msq - mean * mean + EPS)
            for s in range(HS):
                wrows_v[t, pl.ds(s * 16, 16)] = (xs[s] - mean) * rstd * gam[s] + bet[s]

        pltpu.async_copy(wrows_v, out_hbm.at[pl.ds(off, CH)], sem_s)

    issue(0, wrows_a, prows_a, sem_wa, sem_pa)

    def pair(i, carry):
        c0 = 2 * i

        @pl.when(i > 0)
        def _():
            issue(c0 + 1, wrows_b, prows_b, sem_wb, sem_pb, sem_sb)

        @pl.when(i == 0)
        def _():
            issue(c0 + 1, wrows_b, prows_b, sem_wb, sem_pb)

        compute(c0, wrows_a, prows_a, sem_wa, sem_pa, sem_sa)

        @pl.when(i < PAIRS - 1)
        def _():
            issue(c0 + 2, wrows_a, prows_a, sem_wa, sem_pa, sem_sa)

        compute(c0 + 1, wrows_b, prows_b, sem_wb, sem_pb, sem_sb)
        return carry

    lax.fori_loop(0, PAIRS, pair, 0)
    last = base + (ITERS - 2) * CH
    pltpu.make_async_copy(wrows_a, out_hbm.at[pl.ds(last, CH)], sem_sa).wait()
    pltpu.make_async_copy(wrows_b, out_hbm.at[pl.ds(last + CH, CH)], sem_sb).wait()


_fused_call = functools.partial(
    pl.kernel,
    mesh=plsc.VectorSubcoreMesh(core_axis_name="c", subcore_axis_name="s"),
    out_type=jax.ShapeDtypeStruct((TOK, HIDDEN), jnp.float32),
    scratch_types=[
        pltpu.VMEM((PER_W,), jnp.int32),
        pltpu.VMEM((PER_W,), jnp.int32),
        pltpu.VMEM((CH, HIDDEN), jnp.float32),
        pltpu.VMEM((CH, HIDDEN), jnp.float32),
        pltpu.VMEM((CH, HIDDEN), jnp.float32),
        pltpu.VMEM((CH, HIDDEN), jnp.float32),
        pltpu.VMEM((HIDDEN,), jnp.float32),
        pltpu.VMEM((HIDDEN,), jnp.float32),
        pltpu.VMEM_SHARED((2 * S, HIDDEN), jnp.float32),
        pltpu.SemaphoreType.DMA,
        pltpu.SemaphoreType.DMA,
        pltpu.SemaphoreType.DMA,
        pltpu.SemaphoreType.DMA,
        pltpu.SemaphoreType.DMA,
        pltpu.SemaphoreType.DMA,
    ],
)(_fused_body)


def _prep_body(pos_ref, type_ref, tt_ref, ptbl_ref, cidx_ref):
    ptbl_ref[...] = type_ref[...][:, None, :] + pos_ref[...][None, :, :]
    r = lax.broadcasted_iota(jnp.int32, (NROW, CH), 0)
    col = lax.broadcasted_iota(jnp.int32, (NROW, CH), 1)
    pos = lax.rem(r * CH + col, S)
    cidx_ref[...] = tt_ref[...] * S + pos


def kernel(input_ids, token_type_ids, word_emb, pos_emb, type_emb, ln_gamma, ln_beta):
    ids_flat = input_ids.reshape(TOK).astype(jnp.int32)
    tt2d = token_type_ids.reshape(NROW, CH).astype(jnp.int32)

    ptbl, cidx = pl.pallas_call(
        _prep_body,
        out_shape=[
            jax.ShapeDtypeStruct((2, S, HIDDEN), jnp.float32),
            jax.ShapeDtypeStruct((NROW, CH), jnp.int32),
        ],
    )(pos_emb[:S], type_emb, tt2d)

    out = _fused_call(ids_flat, cidx.reshape(TOK), word_emb,
                      ptbl.reshape(2 * S, HIDDEN), ln_gamma, ln_beta)
    return out.reshape(B, S, HIDDEN)
